# Initial kernel scaffold; baseline (speedup 1.0000x reference)
import functools

import jax
import jax.numpy as jnp
from jax import lax
from jax.experimental import pallas as pl
from jax.experimental.pallas import tpu as pltpu
from jax.experimental.pallas import tpu_sc as plsc

N = 10000
E = 320000
D = 128
H = 64
R = 64
L = 4
G = 64
CUTOFF = 6.0
GAMMA = 10.0

N_PAD = 10240
NC, NS = 2, 16
NW = NC * NS
K = 128
E_PAD = 323584
EPW = E_PAD // NW
NB = 1024
NBLK = N_PAD // NB
EB = 1024
EBLK = E_PAD // EB
ROWS_PER_TILE = N_PAD // NS


def _mesh():
  return plsc.VectorSubcoreMesh(
      core_axis_name="c", subcore_axis_name="s", num_cores=NC, num_subcores=NS)


def _make_d2_kernel(interpret=False):
  CH = 512

  @functools.partial(
      pl.kernel, mesh=_mesh(),
      out_type=jax.ShapeDtypeStruct((E_PAD,), jnp.float32),
      scratch_types=[
          pltpu.VMEM((N_PAD,), jnp.float32),
          pltpu.VMEM((N_PAD,), jnp.float32),
          pltpu.VMEM((N_PAD,), jnp.float32),
          pltpu.VMEM((CH,), jnp.int32),
          pltpu.VMEM((CH,), jnp.int32),
          pltpu.VMEM((CH,), jnp.float32),
      ],
      interpret=interpret)
  def d2_kernel(px_hbm, py_hbm, pz_hbm, row_hbm, col_hbm, d2_hbm,
                px_v, py_v, pz_v, row_v, col_v, d2_v):
    wid = lax.axis_index("c") * NS + lax.axis_index("s")
    base = wid * EPW
    pltpu.sync_copy(px_hbm, px_v)
    pltpu.sync_copy(py_hbm, py_v)
    pltpu.sync_copy(pz_hbm, pz_v)

    def step(st, _):
      off = base + st * CH
      pltpu.sync_copy(row_hbm.at[pl.ds(off, CH)], row_v)
      pltpu.sync_copy(col_hbm.at[pl.ds(off, CH)], col_v)

      def inner(j, _):
        ri = row_v[pl.ds(j * 16, 16)]
        ci = col_v[pl.ds(j * 16, 16)]
        dx = plsc.load_gather(px_v, [ri]) - plsc.load_gather(px_v, [ci])
        dy = plsc.load_gather(py_v, [ri]) - plsc.load_gather(py_v, [ci])
        dz = plsc.load_gather(pz_v, [ri]) - plsc.load_gather(pz_v, [ci])
        d2_v[pl.ds(j * 16, 16)] = dx * dx + dy * dy + dz * dz
        return 0

      lax.fori_loop(0, CH // 16, inner, 0)
      pltpu.sync_copy(d2_v, d2_hbm.at[pl.ds(off, CH)])
      return 0

    lax.fori_loop(0, EPW // CH, step, 0)

  return d2_kernel


def _make_edge_kernel(with_deg, interpret=False):
  outs = [jax.ShapeDtypeStruct((NC * N_PAD, H), jnp.float32)]
  if with_deg:
    outs = outs + [jax.ShapeDtypeStruct((NC * N_PAD, H), jnp.float32)]

  @functools.partial(
      pl.kernel, mesh=_mesh(),
      out_type=outs,
      scratch_types=[
          pltpu.VMEM((K,), jnp.int32),
          pltpu.VMEM((K,), jnp.int32),
          pltpu.VMEM((1, K), jnp.int32),
          pltpu.VMEM((K, H), jnp.float32),
          pltpu.VMEM((K, H), jnp.float32),
          pltpu.VMEM((K, H), jnp.float32),
          pltpu.VMEM((K, H), jnp.float32),
          pltpu.VMEM_SHARED((N_PAD, H), jnp.float32),
          pltpu.VMEM_SHARED((N_PAD, H), jnp.float32),
          pltpu.SemaphoreType.DMA,
          pltpu.SemaphoreType.DMA,
      ],
      interpret=interpret)
  def edge_kernel(*args):
    if with_deg:
      (p_hbm, q_hbm, t_hbm, row_hbm, col_hbm, s_hbm, deg_hbm,
       row_v, col_v, cols_v, t_v, gp_v, gq_v, s_v, aggr_s, deg_s,
       sem1, sem2) = args
    else:
      (p_hbm, q_hbm, t_hbm, row_hbm, col_hbm, s_hbm,
       row_v, col_v, cols_v, t_v, gp_v, gq_v, s_v, aggr_s, deg_s,
       sem1, sem2) = args
      deg_hbm = None
    cid = lax.axis_index("c")
    sid = lax.axis_index("s")
    wid = cid * NS + sid
    ebase = wid * EPW

    def zloop(i, _):
      for kk in range(H // 16):
        s_v[i, pl.ds(kk * 16, 16)] = jnp.zeros((16,), jnp.float32)
      return 0
    lax.fori_loop(0, K, zloop, 0)
    for j in range(ROWS_PER_TILE // K):
      pltpu.sync_copy(s_v, aggr_s.at[pl.ds(sid * ROWS_PER_TILE + j * K, K)])
      if with_deg:
        pltpu.sync_copy(s_v, deg_s.at[pl.ds(sid * ROWS_PER_TILE + j * K, K)])
    plsc.subcore_barrier()

    def step(st, _):
      off = ebase + st * K
      pltpu.sync_copy(row_hbm.at[pl.ds(off, K)], row_v)
      pltpu.sync_copy(col_hbm.at[pl.ds(off, K)], col_v)
      pltpu.sync_copy(col_hbm.at[pl.ds(off, K)], cols_v.at[0])
      pltpu.sync_copy(t_hbm.at[pl.ds(off, K)], t_v)
      cp = pltpu.async_copy(p_hbm.at[col_v], gp_v, sem1)
      cq = pltpu.async_copy(q_hbm.at[row_v], gq_v, sem2)
      cp.wait()
      cq.wait()

      def body(i, _):
        for kk in range(H // 16):
          k = kk * 16
          z = (gp_v[i, pl.ds(k, 16)] + gq_v[i, pl.ds(k, 16)]
               + t_v[i, pl.ds(k, 16)])
          s_v[i, pl.ds(k, 16)] = z / (1.0 + jnp.exp(-z))
        return 0

      lax.fori_loop(0, K, body, 0)
      pltpu.sync_copy(s_v, aggr_s.at[cols_v.at[0]], add=True)
      return 0

    lax.fori_loop(0, EPW // K, step, 0)

    if with_deg:
      def oloop(i, _):
        for kk in range(H // 16):
          s_v[i, pl.ds(kk * 16, 16)] = jnp.ones((16,), jnp.float32)
        return 0
      lax.fori_loop(0, K, oloop, 0)

      def dstep(st, _):
        off = ebase + st * K
        pltpu.sync_copy(col_hbm.at[pl.ds(off, K)], cols_v.at[0])
        pltpu.sync_copy(s_v, deg_s.at[cols_v.at[0]], add=True)
        return 0
      lax.fori_loop(0, EPW // K, dstep, 0)

    plsc.subcore_barrier()

    nbase = cid * N_PAD + sid * ROWS_PER_TILE
    for j in range(ROWS_PER_TILE // K):
      pltpu.sync_copy(aggr_s.at[pl.ds(sid * ROWS_PER_TILE + j * K, K)], s_v)
      pltpu.sync_copy(s_v, s_hbm.at[pl.ds(nbase + j * K, K)])
    if with_deg:
      for j in range(ROWS_PER_TILE // K):
        pltpu.sync_copy(deg_s.at[pl.ds(sid * ROWS_PER_TILE + j * K, K)], s_v)
        pltpu.sync_copy(s_v, deg_hbm.at[pl.ds(nbase + j * K, K)])

  return edge_kernel


def _t_body(d2_ref, wc_ref, bt_ref, out_ref):
  d = jnp.sqrt(d2_ref[...])
  c = lax.broadcasted_iota(jnp.float32, (1, R), 1) * (CUTOFF / (R - 1))
  a = d - c
  rbf = jnp.exp(-GAMMA * a * a)
  out_ref[...] = (jnp.dot(rbf, wc_ref[0], preferred_element_type=jnp.float32)
                  + bt_ref[0])


def _t_all_layers(d2, wc, bt, interpret=False):
  return pl.pallas_call(
      _t_body,
      grid=(L, EBLK),
      in_specs=[
          pl.BlockSpec((EB, 1), lambda l, i: (i, 0)),
          pl.BlockSpec((1, R, H), lambda l, i: (l, 0, 0)),
          pl.BlockSpec((1, 1, H), lambda l, i: (l, 0, 0)),
      ],
      out_specs=pl.BlockSpec((EB, H), lambda l, i: (l * EBLK + i, 0)),
      out_shape=jax.ShapeDtypeStruct((L * E_PAD, H), jnp.float32),
      interpret=interpret,
  )(d2, wc, bt)


def _pq_body(h_ref, wp_ref, wq_ref, bp_ref, bq_ref, p_ref, q_ref):
  hb = h_ref[...]
  p_ref[...] = (jnp.dot(hb, wp_ref[...], preferred_element_type=jnp.float32)
                + bp_ref[...])
  q_ref[...] = (jnp.dot(hb, wq_ref[...], preferred_element_type=jnp.float32)
                + bq_ref[...])


def _pq(h, wp, wq, bp, bq, interpret=False):
  return pl.pallas_call(
      _pq_body,
      grid=(NBLK,),
      in_specs=[
          pl.BlockSpec((NB, D), lambda i: (i, 0)),
          pl.BlockSpec((D, H), lambda i: (0, 0)),
          pl.BlockSpec((D, H), lambda i: (0, 0)),
          pl.BlockSpec((1, H), lambda i: (0, 0)),
          pl.BlockSpec((1, H), lambda i: (0, 0)),
      ],
      out_specs=[pl.BlockSpec((NB, H), lambda i: (i, 0)),
                 pl.BlockSpec((NB, H), lambda i: (i, 0))],
      out_shape=[jax.ShapeDtypeStruct((N_PAD, H), jnp.float32),
                 jax.ShapeDtypeStruct((N_PAD, H), jnp.float32)],
      interpret=interpret,
  )(h, wp, wq, bp, bq)


def _silu(v):
  return v * jax.nn.sigmoid(v)


def _upd_body(s0_ref, s1_ref, d0_ref, d1_ref, wu_ref, bv_ref, bu1_ref,
              wu2_ref, bu2_ref, h_ref):
  sb = s0_ref[...] + s1_ref[...]
  deg = d0_ref[...] + d1_ref[...]
  u = _silu(jnp.dot(sb, wu_ref[...], preferred_element_type=jnp.float32)
            + deg * bv_ref[...] + bu1_ref[...])
  h_ref[...] = (jnp.dot(u, wu2_ref[...], preferred_element_type=jnp.float32)
                + bu2_ref[...])


def _node_update(s, deg, wu, bv, bu1, wu2, bu2, interpret=False):
  return pl.pallas_call(
      _upd_body,
      grid=(NBLK,),
      in_specs=[
          pl.BlockSpec((NB, H), lambda i: (i, 0)),
          pl.BlockSpec((NB, H), lambda i: (NBLK + i, 0)),
          pl.BlockSpec((NB, 1), lambda i: (i, 0)),
          pl.BlockSpec((NB, 1), lambda i: (NBLK + i, 0)),
          pl.BlockSpec((H, H), lambda i: (0, 0)),
          pl.BlockSpec((1, H), lambda i: (0, 0)),
          pl.BlockSpec((1, H), lambda i: (0, 0)),
          pl.BlockSpec((H, D), lambda i: (0, 0)),
          pl.BlockSpec((1, D), lambda i: (0, 0)),
      ],
      out_specs=pl.BlockSpec((NB, D), lambda i: (i, 0)),
      out_shape=jax.ShapeDtypeStruct((N_PAD, D), jnp.float32),
      interpret=interpret,
  )(s, s, deg, deg, wu, bv, bu1, wu2, bu2)


def _pool_body(h_ref, b_ref, wf1_ref, bf1_ref, wf2_ref, bf2_ref, o_ref,
               acc_ref):
  i = pl.program_id(0)

  @pl.when(i == 0)
  def _():
    acc_ref[...] = jnp.zeros_like(acc_ref)

  seg = lax.broadcasted_iota(jnp.int32, (G, NB), 0)
  onehot = (seg == b_ref[...].reshape(1, NB)).astype(jnp.float32)
  acc_ref[...] += jnp.dot(onehot, h_ref[...],
                          preferred_element_type=jnp.float32)

  @pl.when(i == NBLK - 1)
  def _():
    g = acc_ref[...]
    s = _silu(jnp.dot(g, wf1_ref[...], preferred_element_type=jnp.float32)
              + bf1_ref[...])
    o_ref[...] = (jnp.sum(s * wf2_ref[...], axis=1, keepdims=True)
                  + bf2_ref[...])


def _pool(h, batch2d, wf1, bf1, wf2row, bf2, interpret=False):
  return pl.pallas_call(
      _pool_body,
      grid=(NBLK,),
      in_specs=[
          pl.BlockSpec((NB, D), lambda i: (i, 0)),
          pl.BlockSpec((NB, 1), lambda i: (i, 0)),
          pl.BlockSpec((D, H), lambda i: (0, 0)),
          pl.BlockSpec((1, H), lambda i: (0, 0)),
          pl.BlockSpec((1, H), lambda i: (0, 0)),
          pl.BlockSpec((1, 1), lambda i: (0, 0)),
      ],
      out_specs=pl.BlockSpec((G, 1), lambda i: (0, 0)),
      out_shape=jax.ShapeDtypeStruct((G, 1), jnp.float32),
      scratch_shapes=[pltpu.VMEM((G, D), jnp.float32)],
      interpret=interpret,
  )(h, batch2d, wf1, bf1, wf2row, bf2)


def _run(x, edge_index, pos, batch, W_node, b_node, W_rbf, b_rbf,
         W_e1, b_e1, W_e2, b_e2, W_u1, b_u1, W_u2, b_u2,
         W_f1, b_f1, W_f2, b_f2, interpret=False):
  f32 = jnp.float32
  A1 = W_e1[:, 0:H, :]
  B1 = W_e1[:, H:2 * H, :]
  C1 = W_e1[:, 2 * H:3 * H, :]
  Wp = jnp.einsum("ldh,lhk->ldk", jnp.broadcast_to(W_node, (L, D, H)), A1)
  Wq = jnp.einsum("ldh,lhk->ldk", jnp.broadcast_to(W_node, (L, D, H)), B1)
  bp = jnp.einsum("lh,lhk->lk", b_node, A1)
  bq = jnp.einsum("lh,lhk->lk", b_node, B1)
  Wc = jnp.einsum("lrh,lhk->lrk", W_rbf, C1)
  bT = (jnp.einsum("lh,lhk->lk", b_rbf, C1) + b_e1)[:, None, :]
  Wu = jnp.einsum("lhk,lkm->lhm", W_e2, W_u1)
  bv = jnp.einsum("lh,lhk->lk", b_e2, W_u1)[:, None, :]
  bu1 = b_u1[:, None, :]
  bu2 = b_u2[:, None, :]

  row = jnp.concatenate([edge_index[0],
                         jnp.full((E_PAD - E,), N, jnp.int32)])
  col = jnp.concatenate([edge_index[1],
                         jnp.full((E_PAD - E,), N, jnp.int32)])
  posp = jnp.concatenate([pos.astype(f32),
                          jnp.zeros((N_PAD - N, 3), f32)], axis=0)
  px = jnp.asarray(posp[:, 0])
  py = jnp.asarray(posp[:, 1])
  pz = jnp.asarray(posp[:, 2])
  xp = jnp.concatenate([x, jnp.zeros((N_PAD - N, D), f32)], axis=0)
  batchp = jnp.concatenate([batch.astype(jnp.int32),
                            jnp.full((N_PAD - N,), G, jnp.int32)])[:, None]

  d2k = _make_d2_kernel(interpret=interpret)
  d2 = d2k(px, py, pz, row, col)
  t_all = _t_all_layers(d2[:, None], Wc, bT, interpret=interpret)

  ek_deg = _make_edge_kernel(True, interpret=interpret)
  ek = _make_edge_kernel(False, interpret=interpret)

  h = xp
  deg = None
  for l in range(L):
    p, q = _pq(h, Wp[l], Wq[l], bp[l][None, :], bq[l][None, :],
               interpret=interpret)
    t_l = lax.slice_in_dim(t_all, l * E_PAD, (l + 1) * E_PAD, axis=0)
    if l == 0:
      s, deg64 = ek_deg(p, q, t_l, row, col)
      deg = deg64[:, :1]
    else:
      (s,) = ek(p, q, t_l, row, col)
    h = _node_update(s, deg, Wu[l], bv[l], bu1[l], W_u2[l], bu2[l],
                     interpret=interpret)

  o = _pool(h, batchp, W_f1, b_f1[None, :], W_f2.T, b_f2[None, :],
            interpret=interpret)
  return o.reshape(-1)


def kernel(x, edge_index, pos, batch, W_node, b_node, W_rbf, b_rbf,
           W_e1, b_e1, W_e2, b_e2, W_u1, b_u1, W_u2, b_u2,
           W_f1, b_f1, W_f2, b_f2):
  return _run(x, edge_index, pos, batch, W_node, b_node, W_rbf, b_rbf,
              W_e1, b_e1, W_e2, b_e2, W_u1, b_u1, W_u2, b_u2,
              W_f1, b_f1, W_f2, b_f2)


# speedup vs baseline: 2.4595x; 2.4595x over previous
import functools

import jax
import jax.numpy as jnp
from jax import lax
from jax.experimental import pallas as pl
from jax.experimental.pallas import tpu as pltpu
from jax.experimental.pallas import tpu_sc as plsc

N = 10000
E = 320000
D = 128
H = 64
R = 64
L = 4
G = 64
CUTOFF = 6.0
GAMMA = 10.0

N_PAD = 10240
NC, NS = 2, 16
NW = NC * NS
K = 128
E_PAD = 323584
EPW = E_PAD // NW
NB = 1024
NBLK = N_PAD // NB
EB = 1024
EBLK = E_PAD // EB
ROWS_PER_TILE = N_PAD // NS
D2CH = 512


def _mesh():
  return plsc.VectorSubcoreMesh(
      core_axis_name="c", subcore_axis_name="s", num_cores=NC, num_subcores=NS)


def _make_d2_kernel(interpret=False):
  CH = D2CH

  @functools.partial(
      pl.kernel, mesh=_mesh(),
      out_type=jax.ShapeDtypeStruct((E_PAD,), jnp.float32),
      scratch_types=[
          pltpu.VMEM((N_PAD,), jnp.float32),
          pltpu.VMEM((N_PAD,), jnp.float32),
          pltpu.VMEM((N_PAD,), jnp.float32),
          pltpu.VMEM((CH,), jnp.int32),
          pltpu.VMEM((CH,), jnp.int32),
          pltpu.VMEM((CH,), jnp.float32),
      ],
      compiler_params=pltpu.CompilerParams(needs_layout_passes=False),
      interpret=interpret)
  def d2_kernel(px_hbm, py_hbm, pz_hbm, row_hbm, col_hbm, d2_hbm,
                px_v, py_v, pz_v, row_v, col_v, d2_v):
    wid = lax.axis_index("c") * NS + lax.axis_index("s")
    base = wid * EPW
    pltpu.sync_copy(px_hbm, px_v)
    pltpu.sync_copy(py_hbm, py_v)
    pltpu.sync_copy(pz_hbm, pz_v)

    def step(st, _):
      off = base + st * CH
      pltpu.sync_copy(row_hbm.at[pl.ds(off, CH)], row_v)
      pltpu.sync_copy(col_hbm.at[pl.ds(off, CH)], col_v)

      def inner(j, _):
        ri = row_v[pl.ds(j * 16, 16)]
        ci = col_v[pl.ds(j * 16, 16)]
        dx = plsc.load_gather(px_v, [ri]) - plsc.load_gather(px_v, [ci])
        dy = plsc.load_gather(py_v, [ri]) - plsc.load_gather(py_v, [ci])
        dz = plsc.load_gather(pz_v, [ri]) - plsc.load_gather(pz_v, [ci])
        d2_v[pl.ds(j * 16, 16)] = dx * dx + dy * dy + dz * dz
        return 0

      lax.fori_loop(0, CH // 16, inner, 0)
      pltpu.sync_copy(d2_v, d2_hbm.at[pl.ds(off, CH)])
      return 0

    lax.fori_loop(0, EPW // CH, step, 0)

  return d2_kernel


def _make_edge_kernel(with_deg, interpret=False):
  outs = [jax.ShapeDtypeStruct((NC * N_PAD, H), jnp.float32)]
  if with_deg:
    outs = outs + [jax.ShapeDtypeStruct((NC * N_PAD, H), jnp.float32)]

  @functools.partial(
      pl.kernel, mesh=_mesh(),
      out_type=outs,
      scratch_types=[
          pltpu.VMEM((K,), jnp.int32),
          pltpu.VMEM((K,), jnp.int32),
          pltpu.VMEM((1, K), jnp.int32),
          pltpu.VMEM((K, H), jnp.float32),
          pltpu.VMEM((K, H), jnp.float32),
          pltpu.VMEM((K, H), jnp.float32),
          pltpu.VMEM((K, H), jnp.float32),
          pltpu.VMEM_SHARED((N_PAD, H), jnp.float32),
          pltpu.VMEM_SHARED((N_PAD, H), jnp.float32),
          pltpu.SemaphoreType.DMA,
          pltpu.SemaphoreType.DMA,
      ],
      compiler_params=pltpu.CompilerParams(needs_layout_passes=False,
                                           use_tc_tiling_on_sc=False),
      interpret=interpret)
  def edge_kernel(*args):
    if with_deg:
      (p_hbm, q_hbm, t_hbm, row_hbm, col_hbm, s_hbm, deg_hbm,
       row_v, col_v, cols_v, t_v, gp_v, gq_v, s_v, aggr_s, deg_s,
       sem1, sem2) = args
    else:
      (p_hbm, q_hbm, t_hbm, row_hbm, col_hbm, s_hbm,
       row_v, col_v, cols_v, t_v, gp_v, gq_v, s_v, aggr_s, deg_s,
       sem1, sem2) = args
      deg_hbm = None
    cid = lax.axis_index("c")
    sid = lax.axis_index("s")
    wid = cid * NS + sid
    ebase = wid * EPW

    def zloop(i, _):
      for kk in range(H // 16):
        s_v[i, pl.ds(kk * 16, 16)] = jnp.zeros((16,), jnp.float32)
      return 0
    lax.fori_loop(0, K, zloop, 0)
    for j in range(ROWS_PER_TILE // K):
      pltpu.sync_copy(s_v, aggr_s.at[pl.ds(sid * ROWS_PER_TILE + j * K, K)])
      if with_deg:
        pltpu.sync_copy(s_v, deg_s.at[pl.ds(sid * ROWS_PER_TILE + j * K, K)])
    plsc.subcore_barrier()

    def step(st, _):
      off = ebase + st * K
      pltpu.sync_copy(row_hbm.at[pl.ds(off, K)], row_v)
      pltpu.sync_copy(col_hbm.at[pl.ds(off, K)], col_v)
      pltpu.sync_copy(col_hbm.at[pl.ds(off, K)], cols_v.at[0])
      pltpu.sync_copy(t_hbm.at[pl.ds(off, K)], t_v)
      cp = pltpu.async_copy(p_hbm.at[col_v], gp_v, sem1)
      cq = pltpu.async_copy(q_hbm.at[row_v], gq_v, sem2)
      cp.wait()
      cq.wait()

      def body(i, _):
        for kk in range(H // 16):
          k = kk * 16
          z = (gp_v[i, pl.ds(k, 16)] + gq_v[i, pl.ds(k, 16)]
               + t_v[i, pl.ds(k, 16)])
          s_v[i, pl.ds(k, 16)] = z / (1.0 + jnp.exp(-z))
        return 0

      lax.fori_loop(0, K, body, 0)
      pltpu.sync_copy(s_v, aggr_s.at[cols_v.at[0]], add=True)
      return 0

    lax.fori_loop(0, EPW // K, step, 0)

    if with_deg:
      def oloop(i, _):
        for kk in range(H // 16):
          s_v[i, pl.ds(kk * 16, 16)] = jnp.ones((16,), jnp.float32)
        return 0
      lax.fori_loop(0, K, oloop, 0)

      def dstep(st, _):
        off = ebase + st * K
        pltpu.sync_copy(col_hbm.at[pl.ds(off, K)], cols_v.at[0])
        pltpu.sync_copy(s_v, deg_s.at[cols_v.at[0]], add=True)
        return 0
      lax.fori_loop(0, EPW // K, dstep, 0)

    plsc.subcore_barrier()

    nbase = cid * N_PAD + sid * ROWS_PER_TILE
    for j in range(ROWS_PER_TILE // K):
      pltpu.sync_copy(aggr_s.at[pl.ds(sid * ROWS_PER_TILE + j * K, K)], s_v)
      pltpu.sync_copy(s_v, s_hbm.at[pl.ds(nbase + j * K, K)])
    if with_deg:
      for j in range(ROWS_PER_TILE // K):
        pltpu.sync_copy(deg_s.at[pl.ds(sid * ROWS_PER_TILE + j * K, K)], s_v)
        pltpu.sync_copy(s_v, deg_hbm.at[pl.ds(nbase + j * K, K)])

  return edge_kernel


def _t_body(d2_ref, wc_ref, bt_ref, out_ref):
  d = jnp.sqrt(d2_ref[...])
  c = (lax.broadcasted_iota(jnp.int32, (1, R), 1).astype(jnp.float32)
       * (CUTOFF / (R - 1)))
  a = d - c
  rbf = jnp.exp(-GAMMA * a * a)
  out_ref[...] = (jnp.dot(rbf, wc_ref[0], preferred_element_type=jnp.float32)
                  + bt_ref[0])


def _t_all_layers(d2, wc, bt, interpret=False):
  return pl.pallas_call(
      _t_body,
      grid=(L, EBLK),
      in_specs=[
          pl.BlockSpec((EB, 1), lambda l, i: (i, 0)),
          pl.BlockSpec((1, R, H), lambda l, i: (l, 0, 0)),
          pl.BlockSpec((1, 1, H), lambda l, i: (l, 0, 0)),
      ],
      out_specs=pl.BlockSpec((EB, H), lambda l, i: (l * EBLK + i, 0)),
      out_shape=jax.ShapeDtypeStruct((L * E_PAD, H), jnp.float32),
      interpret=interpret,
  )(d2, wc, bt)


def _pq_body(h_ref, wp_ref, wq_ref, bp_ref, bq_ref, p_ref, q_ref):
  hb = h_ref[...]
  p_ref[...] = (jnp.dot(hb, wp_ref[...], preferred_element_type=jnp.float32)
                + bp_ref[...])
  q_ref[...] = (jnp.dot(hb, wq_ref[...], preferred_element_type=jnp.float32)
                + bq_ref[...])


def _pq(h, wp, wq, bp, bq, interpret=False):
  return pl.pallas_call(
      _pq_body,
      grid=(NBLK,),
      in_specs=[
          pl.BlockSpec((NB, D), lambda i: (i, 0)),
          pl.BlockSpec((D, H), lambda i: (0, 0)),
          pl.BlockSpec((D, H), lambda i: (0, 0)),
          pl.BlockSpec((1, H), lambda i: (0, 0)),
          pl.BlockSpec((1, H), lambda i: (0, 0)),
      ],
      out_specs=[pl.BlockSpec((NB, H), lambda i: (i, 0)),
                 pl.BlockSpec((NB, H), lambda i: (i, 0))],
      out_shape=[jax.ShapeDtypeStruct((N_PAD, H), jnp.float32),
                 jax.ShapeDtypeStruct((N_PAD, H), jnp.float32)],
      interpret=interpret,
  )(h, wp, wq, bp, bq)


def _silu(v):
  return v * jax.nn.sigmoid(v)


def _upd_body(s0_ref, s1_ref, d0_ref, d1_ref, wu_ref, bv_ref, bu1_ref,
              wu2_ref, bu2_ref, h_ref):
  sb = s0_ref[...] + s1_ref[...]
  deg = d0_ref[...] + d1_ref[...]
  u = _silu(jnp.dot(sb, wu_ref[...], preferred_element_type=jnp.float32)
            + deg * bv_ref[...] + bu1_ref[...])
  h_ref[...] = (jnp.dot(u, wu2_ref[...], preferred_element_type=jnp.float32)
                + bu2_ref[...])


def _node_update(s, deg, wu, bv, bu1, wu2, bu2, interpret=False):
  return pl.pallas_call(
      _upd_body,
      grid=(NBLK,),
      in_specs=[
          pl.BlockSpec((NB, H), lambda i: (i, 0)),
          pl.BlockSpec((NB, H), lambda i: (NBLK + i, 0)),
          pl.BlockSpec((NB, 1), lambda i: (i, 0)),
          pl.BlockSpec((NB, 1), lambda i: (NBLK + i, 0)),
          pl.BlockSpec((H, H), lambda i: (0, 0)),
          pl.BlockSpec((1, H), lambda i: (0, 0)),
          pl.BlockSpec((1, H), lambda i: (0, 0)),
          pl.BlockSpec((H, D), lambda i: (0, 0)),
          pl.BlockSpec((1, D), lambda i: (0, 0)),
      ],
      out_specs=pl.BlockSpec((NB, D), lambda i: (i, 0)),
      out_shape=jax.ShapeDtypeStruct((N_PAD, D), jnp.float32),
      interpret=interpret,
  )(s, s, deg, deg, wu, bv, bu1, wu2, bu2)


def _pool_body(h_ref, b_ref, wf1_ref, bf1_ref, wf2_ref, bf2_ref, o_ref,
               acc_ref):
  i = pl.program_id(0)

  @pl.when(i == 0)
  def _():
    acc_ref[...] = jnp.zeros_like(acc_ref)

  seg = lax.broadcasted_iota(jnp.int32, (G, NB), 0)
  onehot = (seg == b_ref[...].reshape(1, NB)).astype(jnp.float32)
  acc_ref[...] += jnp.dot(onehot, h_ref[...],
                          preferred_element_type=jnp.float32)

  @pl.when(i == NBLK - 1)
  def _():
    g = acc_ref[...]
    s = _silu(jnp.dot(g, wf1_ref[...], preferred_element_type=jnp.float32)
              + bf1_ref[...])
    o_ref[...] = (jnp.sum(s * wf2_ref[...], axis=1, keepdims=True)
                  + bf2_ref[...])


def _pool(h, batch2d, wf1, bf1, wf2row, bf2, interpret=False):
  return pl.pallas_call(
      _pool_body,
      grid=(NBLK,),
      in_specs=[
          pl.BlockSpec((NB, D), lambda i: (i, 0)),
          pl.BlockSpec((NB, 1), lambda i: (i, 0)),
          pl.BlockSpec((D, H), lambda i: (0, 0)),
          pl.BlockSpec((1, H), lambda i: (0, 0)),
          pl.BlockSpec((1, H), lambda i: (0, 0)),
          pl.BlockSpec((1, 1), lambda i: (0, 0)),
      ],
      out_specs=pl.BlockSpec((G, 1), lambda i: (0, 0)),
      out_shape=jax.ShapeDtypeStruct((G, 1), jnp.float32),
      scratch_shapes=[pltpu.VMEM((G, D), jnp.float32)],
      interpret=interpret,
  )(h, batch2d, wf1, bf1, wf2row, bf2)


def _run(x, edge_index, pos, batch, W_node, b_node, W_rbf, b_rbf,
         W_e1, b_e1, W_e2, b_e2, W_u1, b_u1, W_u2, b_u2,
         W_f1, b_f1, W_f2, b_f2, interpret=False):
  f32 = jnp.float32
  A1 = W_e1[:, 0:H, :]
  B1 = W_e1[:, H:2 * H, :]
  C1 = W_e1[:, 2 * H:3 * H, :]
  Wp = jnp.einsum("ldh,lhk->ldk", jnp.broadcast_to(W_node, (L, D, H)), A1)
  Wq = jnp.einsum("ldh,lhk->ldk", jnp.broadcast_to(W_node, (L, D, H)), B1)
  bp = jnp.einsum("lh,lhk->lk", b_node, A1)
  bq = jnp.einsum("lh,lhk->lk", b_node, B1)
  Wc = jnp.einsum("lrh,lhk->lrk", W_rbf, C1)
  bT = (jnp.einsum("lh,lhk->lk", b_rbf, C1) + b_e1)[:, None, :]
  Wu = jnp.einsum("lhk,lkm->lhm", W_e2, W_u1)
  bv = jnp.einsum("lh,lhk->lk", b_e2, W_u1)[:, None, :]
  bu1 = b_u1[:, None, :]
  bu2 = b_u2[:, None, :]

  row = jnp.concatenate([edge_index[0],
                         jnp.full((E_PAD - E,), N, jnp.int32)])
  col = jnp.concatenate([edge_index[1],
                         jnp.full((E_PAD - E,), N, jnp.int32)])
  posp = jnp.concatenate([pos.astype(f32),
                          jnp.zeros((N_PAD - N, 3), f32)], axis=0)
  px = jnp.asarray(posp[:, 0])
  py = jnp.asarray(posp[:, 1])
  pz = jnp.asarray(posp[:, 2])
  xp = jnp.concatenate([x, jnp.zeros((N_PAD - N, D), f32)], axis=0)
  batchp = jnp.concatenate([batch.astype(jnp.int32),
                            jnp.full((N_PAD - N,), G, jnp.int32)])[:, None]

  d2k = _make_d2_kernel(interpret=interpret)
  d2 = d2k(px, py, pz, row, col)
  t_all = _t_all_layers(d2[:, None], Wc, bT, interpret=interpret)

  ek_deg = _make_edge_kernel(True, interpret=interpret)
  ek = _make_edge_kernel(False, interpret=interpret)

  h = xp
  deg = None
  for l in range(L):
    p, q = _pq(h, Wp[l], Wq[l], bp[l][None, :], bq[l][None, :],
               interpret=interpret)
    t_l = lax.slice_in_dim(t_all, l * E_PAD, (l + 1) * E_PAD, axis=0)
    if l == 0:
      s, deg64 = ek_deg(p, q, t_l, row, col)
      deg = deg64[:, :1]
    else:
      (s,) = ek(p, q, t_l, row, col)
    h = _node_update(s, deg, Wu[l], bv[l], bu1[l], W_u2[l], bu2[l],
                     interpret=interpret)

  o = _pool(h, batchp, W_f1, b_f1[None, :], W_f2.T, b_f2[None, :],
            interpret=interpret)
  return o.reshape(-1)


def kernel(x, edge_index, pos, batch, W_node, b_node, W_rbf, b_rbf,
           W_e1, b_e1, W_e2, b_e2, W_u1, b_u1, W_u2, b_u2,
           W_f1, b_f1, W_f2, b_f2):
  return _run(x, edge_index, pos, batch, W_node, b_node, W_rbf, b_rbf,
              W_e1, b_e1, W_e2, b_e2, W_u1, b_u1, W_u2, b_u2,
              W_f1, b_f1, W_f2, b_f2)
